# Initial kernel scaffold; baseline (speedup 1.0000x reference)
import jax
import jax.numpy as jnp
from jax.experimental import pallas as pl


def kernel(inputs, embeddings):
    raise NotImplementedError("write your pallas kernel here")


# speedup vs baseline: 1.1014x; 1.1014x over previous
import functools

import jax
import jax.numpy as jnp
from jax import lax
from jax.experimental import pallas as pl
from jax.experimental.pallas import tpu as pltpu
from jax.experimental.pallas import tpu_sc as plsc

M = 16384
K = 8192
D = 256
TM = 256
MT = M // TM
TK = 512
KT = K // TK

COMMIT = 0.25
LOSS_SCALE = (1.0 + COMMIT) / (M * D)

NC = 2
NS = 16
NW = NC * NS
BPW = M // NW
CHUNK = 256


def _row_sumsq(v):
    sq = v * v
    h = sq[:, :128] + sq[:, 128:]
    acc = h[:, 0:8]
    for t in range(1, 16):
        acc = acc + h[:, 8 * t:8 * t + 8]
    a4 = acc[:, 0:4] + acc[:, 4:8]
    a2 = a4[:, 0:2] + a4[:, 2:4]
    return (a2[:, 0:1] + a2[:, 1:2])[:, 0]


def _e2_body(e_ref, o_ref):
    o_ref[0, 0, :] = _row_sumsq(e_ref[...])


@jax.jit
def _codebook_norms(embeddings):
    return pl.pallas_call(
        _e2_body,
        grid=(KT,),
        in_specs=[pl.BlockSpec((TK, D), lambda k: (k, 0))],
        out_specs=pl.BlockSpec((1, 1, TK), lambda k: (k, 0, 0)),
        out_shape=jax.ShapeDtypeStruct((KT, 1, TK), jnp.float32),
    )(embeddings)


def _dist_body(x_ref, e_ref, e2_ref, idx_ref, loss_ref):
    m = pl.program_id(0)

    @pl.when(m == 0)
    def _():
        loss_ref[0, 0] = 0.0

    e = e_ref[...]
    x = x_ref[...]
    xe = lax.dot_general(x, e, (((1,), (1,)), ((), ())),
                         preferred_element_type=jnp.float32)
    xn = _row_sumsq(x)
    dist = (xn[:, None] + e2_ref[0:1, :]) - 2.0 * xe

    minv = jnp.min(dist, axis=1, keepdims=True)
    iota = lax.broadcasted_iota(jnp.int32, (TM, K), 1)
    cand = jnp.where(dist == minv, iota, K)
    idx = jnp.min(cand, axis=1, keepdims=True)
    idx_ref[0, 0, :] = idx[:, 0]

    rowloss = minv[:, 0]
    loss_ref[0, 0] += jnp.sum(rowloss) * LOSS_SCALE


@jax.jit
def _dist_argmin(flat_x, embeddings, e2):
    return pl.pallas_call(
        _dist_body,
        grid=(MT,),
        in_specs=[
            pl.BlockSpec((TM, D), lambda m: (m, 0)),
            pl.BlockSpec((K, D), lambda m: (0, 0)),
            pl.BlockSpec((1, K), lambda m: (0, 0)),
        ],
        out_specs=[
            pl.BlockSpec((1, 1, TM), lambda m: (m, 0, 0)),
            pl.BlockSpec(memory_space=pltpu.SMEM),
        ],
        out_shape=[
            jax.ShapeDtypeStruct((MT, 1, TM), jnp.int32),
            jax.ShapeDtypeStruct((1, 1), jnp.float32),
        ],
    )(flat_x, embeddings, e2)


def _gather_body(table_hbm, idx_hbm, out_hbm, idx_v, rows_v, sem):
    wid = lax.axis_index("s") * NC + lax.axis_index("c")
    for c in range(BPW // CHUNK):
        base = wid * BPW + c * CHUNK
        pltpu.sync_copy(idx_hbm.at[pl.ds(base, CHUNK)], idx_v)
        pltpu.async_copy(table_hbm.at[idx_v], rows_v, sem).wait()
        pltpu.sync_copy(rows_v, out_hbm.at[pl.ds(base, CHUNK)])


@jax.jit
def _sc_gather(embeddings, idx):
    return pl.kernel(
        _gather_body,
        out_type=jax.ShapeDtypeStruct((M, D), jnp.float32),
        mesh=plsc.VectorSubcoreMesh(core_axis_name="c", subcore_axis_name="s"),
        scratch_types=[
            pltpu.VMEM((CHUNK,), jnp.int32),
            pltpu.VMEM((CHUNK, D), jnp.float32),
            pltpu.SemaphoreType.DMA,
        ],
    )(embeddings, idx)


def kernel(inputs, embeddings):
    flat_x = inputs.reshape(M, D)
    e2 = _codebook_norms(embeddings).reshape(1, K)
    idx3, loss = _dist_argmin(flat_x, embeddings, e2)
    idx = idx3.reshape(M)
    quantized = _sc_gather(embeddings, idx)
    return (quantized.reshape(inputs.shape), loss[0, 0], idx[:, None])


# speedup vs baseline: 1.1822x; 1.0733x over previous
import functools

import jax
import jax.numpy as jnp
from jax import lax
from jax.experimental import pallas as pl
from jax.experimental.pallas import tpu as pltpu
from jax.experimental.pallas import tpu_sc as plsc

M = 16384
K = 8192
D = 256
TM = 512
MT = M // TM
TK = 512
KT = K // TK

COMMIT = 0.25
LOSS_SCALE = (1.0 + COMMIT) / (M * D)

NC = 2
NS = 16
NW = NC * NS
BPW = M // NW
CHUNK = 256


def _row_sumsq(v):
    sq = v * v
    h = sq[:, :128] + sq[:, 128:]
    acc = h[:, 0:8]
    for t in range(1, 16):
        acc = acc + h[:, 8 * t:8 * t + 8]
    a4 = acc[:, 0:4] + acc[:, 4:8]
    a2 = a4[:, 0:2] + a4[:, 2:4]
    return (a2[:, 0:1] + a2[:, 1:2])[:, 0]


def _e2_body(e_ref, o_ref):
    o_ref[0, 0, :] = _row_sumsq(e_ref[...])


@jax.jit
def _codebook_norms(embeddings):
    return pl.pallas_call(
        _e2_body,
        grid=(KT,),
        in_specs=[pl.BlockSpec((TK, D), lambda k: (k, 0))],
        out_specs=pl.BlockSpec((1, 1, TK), lambda k: (k, 0, 0)),
        out_shape=jax.ShapeDtypeStruct((KT, 1, TK), jnp.float32),
    )(embeddings)


def _dist_body(x_ref, e_ref, e2_ref, idx_ref, loss_ref):
    m = pl.program_id(0)

    @pl.when(m == 0)
    def _():
        loss_ref[0, 0] = 0.0

    e = e_ref[...]
    x = x_ref[...]
    xe = lax.dot_general(x, e, (((1,), (1,)), ((), ())),
                         preferred_element_type=jnp.float32)
    xn = _row_sumsq(x)
    dist = (xn[:, None] + e2_ref[0:1, :]) - 2.0 * xe

    minv = jnp.min(dist, axis=1, keepdims=True)
    iota = lax.broadcasted_iota(jnp.int32, (TM, K), 1)
    cand = jnp.where(dist == minv, iota, K)
    idx = jnp.min(cand, axis=1, keepdims=True)
    idx_ref[0, 0, :] = idx[:, 0]

    rowloss = minv[:, 0]
    loss_ref[0, 0] += jnp.sum(rowloss) * LOSS_SCALE


@jax.jit
def _dist_argmin(flat_x, embeddings, e2):
    return pl.pallas_call(
        _dist_body,
        grid=(MT,),
        in_specs=[
            pl.BlockSpec((TM, D), lambda m: (m, 0)),
            pl.BlockSpec((K, D), lambda m: (0, 0)),
            pl.BlockSpec((1, K), lambda m: (0, 0)),
        ],
        out_specs=[
            pl.BlockSpec((1, 1, TM), lambda m: (m, 0, 0)),
            pl.BlockSpec(memory_space=pltpu.SMEM),
        ],
        out_shape=[
            jax.ShapeDtypeStruct((MT, 1, TM), jnp.int32),
            jax.ShapeDtypeStruct((1, 1), jnp.float32),
        ],
    )(flat_x, embeddings, e2)


def _gather_body(table_hbm, idx_hbm, out_hbm, idx_v, rows_v, sem):
    wid = lax.axis_index("s") * NC + lax.axis_index("c")
    for c in range(BPW // CHUNK):
        base = wid * BPW + c * CHUNK
        pltpu.sync_copy(idx_hbm.at[pl.ds(base, CHUNK)], idx_v)
        pltpu.async_copy(table_hbm.at[idx_v], rows_v, sem).wait()
        pltpu.sync_copy(rows_v, out_hbm.at[pl.ds(base, CHUNK)])


@jax.jit
def _sc_gather(embeddings, idx):
    return pl.kernel(
        _gather_body,
        out_type=jax.ShapeDtypeStruct((M, D), jnp.float32),
        mesh=plsc.VectorSubcoreMesh(core_axis_name="c", subcore_axis_name="s"),
        scratch_types=[
            pltpu.VMEM((CHUNK,), jnp.int32),
            pltpu.VMEM((CHUNK, D), jnp.float32),
            pltpu.SemaphoreType.DMA,
        ],
    )(embeddings, idx)


def kernel(inputs, embeddings):
    flat_x = inputs.reshape(M, D)
    e2 = _codebook_norms(embeddings).reshape(1, K)
    idx3, loss = _dist_argmin(flat_x, embeddings, e2)
    idx = idx3.reshape(M)
    quantized = _sc_gather(embeddings, idx)
    return (quantized.reshape(inputs.shape), loss[0, 0], idx[:, None])


# speedup vs baseline: 1.2183x; 1.0306x over previous
import functools

import jax
import jax.numpy as jnp
from jax import lax
from jax.experimental import pallas as pl
from jax.experimental.pallas import tpu as pltpu
from jax.experimental.pallas import tpu_sc as plsc

M = 16384
K = 8192
D = 256
TM = 1024
MT = M // TM
TK = 512
KT = K // TK

COMMIT = 0.25
LOSS_SCALE = (1.0 + COMMIT) / (M * D)

NC = 2
NS = 16
NW = NC * NS
BPW = M // NW
CHUNK = 256


def _row_sumsq(v):
    sq = v * v
    h = sq[:, :128] + sq[:, 128:]
    acc = h[:, 0:8]
    for t in range(1, 16):
        acc = acc + h[:, 8 * t:8 * t + 8]
    a4 = acc[:, 0:4] + acc[:, 4:8]
    a2 = a4[:, 0:2] + a4[:, 2:4]
    return (a2[:, 0:1] + a2[:, 1:2])[:, 0]


def _e2_body(e_ref, o_ref):
    o_ref[0, 0, :] = _row_sumsq(e_ref[...])


@jax.jit
def _codebook_norms(embeddings):
    return pl.pallas_call(
        _e2_body,
        grid=(KT,),
        in_specs=[pl.BlockSpec((TK, D), lambda k: (k, 0))],
        out_specs=pl.BlockSpec((1, 1, TK), lambda k: (k, 0, 0)),
        out_shape=jax.ShapeDtypeStruct((KT, 1, TK), jnp.float32),
    )(embeddings)


def _dist_body(x_ref, e_ref, e2_ref, idx_ref, loss_ref):
    m = pl.program_id(0)

    @pl.when(m == 0)
    def _():
        loss_ref[0, 0] = 0.0

    e = e_ref[...]
    x = x_ref[...]
    xe = lax.dot_general(x, e, (((1,), (1,)), ((), ())),
                         preferred_element_type=jnp.float32)
    xn = _row_sumsq(x)
    dist = (xn[:, None] + e2_ref[0:1, :]) - 2.0 * xe

    minv = jnp.min(dist, axis=1, keepdims=True)
    iota = lax.broadcasted_iota(jnp.int32, (TM, K), 1)
    cand = jnp.where(dist == minv, iota, K)
    idx = jnp.min(cand, axis=1, keepdims=True)
    idx_ref[0, 0, :] = idx[:, 0]

    rowloss = minv[:, 0]
    loss_ref[0, 0] += jnp.sum(rowloss) * LOSS_SCALE


@jax.jit
def _dist_argmin(flat_x, embeddings, e2):
    return pl.pallas_call(
        _dist_body,
        grid=(MT,),
        in_specs=[
            pl.BlockSpec((TM, D), lambda m: (m, 0)),
            pl.BlockSpec((K, D), lambda m: (0, 0)),
            pl.BlockSpec((1, K), lambda m: (0, 0)),
        ],
        out_specs=[
            pl.BlockSpec((1, 1, TM), lambda m: (m, 0, 0)),
            pl.BlockSpec(memory_space=pltpu.SMEM),
        ],
        out_shape=[
            jax.ShapeDtypeStruct((MT, 1, TM), jnp.int32),
            jax.ShapeDtypeStruct((1, 1), jnp.float32),
        ],
    )(flat_x, embeddings, e2)


def _gather_body(table_hbm, idx_hbm, out_hbm, idx_v, rows_v, sem):
    wid = lax.axis_index("s") * NC + lax.axis_index("c")
    for c in range(BPW // CHUNK):
        base = wid * BPW + c * CHUNK
        pltpu.sync_copy(idx_hbm.at[pl.ds(base, CHUNK)], idx_v)
        pltpu.async_copy(table_hbm.at[idx_v], rows_v, sem).wait()
        pltpu.sync_copy(rows_v, out_hbm.at[pl.ds(base, CHUNK)])


@jax.jit
def _sc_gather(embeddings, idx):
    return pl.kernel(
        _gather_body,
        out_type=jax.ShapeDtypeStruct((M, D), jnp.float32),
        mesh=plsc.VectorSubcoreMesh(core_axis_name="c", subcore_axis_name="s"),
        scratch_types=[
            pltpu.VMEM((CHUNK,), jnp.int32),
            pltpu.VMEM((CHUNK, D), jnp.float32),
            pltpu.SemaphoreType.DMA,
        ],
    )(embeddings, idx)


def kernel(inputs, embeddings):
    flat_x = inputs.reshape(M, D)
    e2 = _codebook_norms(embeddings).reshape(1, K)
    idx3, loss = _dist_argmin(flat_x, embeddings, e2)
    idx = idx3.reshape(M)
    quantized = _sc_gather(embeddings, idx)
    return (quantized.reshape(inputs.shape), loss[0, 0], idx[:, None])
